# baseline (speedup 1.0000x reference)
import functools

import jax
import jax.numpy as jnp
import numpy as np
from jax import lax
from jax.experimental import pallas as pl
from jax.experimental.pallas import tpu as pltpu
from jax.experimental.pallas import tpu_sc as plsc

_RATIO = 0.75
_T, _B, _C = 576, 64, 768
_REMAIN_T = int(_T * (1 - _RATIO))

_NC, _NS = 2, 16
_NW = _NC * _NS
_ROWS = _REMAIN_T * _B
_ROWS_PER_W = _ROWS // _NW
_CHUNK = 24
_NCHUNK = _ROWS_PER_W // _CHUNK
_NBUF = 6


def _make_indexes():
    with jax.default_device(jax.local_devices(backend="cpu")[0]):
        key = jax.random.key(42)
        keys = jax.random.split(key, _B)
        fwd = jnp.stack([jax.random.permutation(k, _T) for k in keys], axis=1)
        bwd = jnp.argsort(fwd, axis=0)
        return np.asarray(fwd), np.asarray(bwd)


_FWD, _BWD = _make_indexes()
_FLAT_IDX = (
    _FWD[:_REMAIN_T].astype(np.int64) * _B + np.arange(_B, dtype=np.int64)[None, :]
).astype(np.int32).reshape(_NW, _NCHUNK, _CHUNK)
_FWDBWD = np.stack([_FWD, _BWD])

@functools.lru_cache(maxsize=1)
def _build_gather_kernel():
    mesh = plsc.VectorSubcoreMesh(
        core_axis_name="c", subcore_axis_name="s",
        num_cores=_NC, num_subcores=_NS,
    )

    @functools.partial(
        pl.kernel,
        out_type=(
            jax.ShapeDtypeStruct((_ROWS, _C), jnp.float32),
            jax.ShapeDtypeStruct((_T, _B), jnp.int32),
            jax.ShapeDtypeStruct((_T, _B), jnp.int32),
        ),
        mesh=mesh,
        scratch_types=[
            pltpu.VMEM((_NCHUNK, _CHUNK), jnp.int32),
        ]
        + [pltpu.VMEM((_CHUNK, _C), jnp.float32)] * _NBUF
        + [pltpu.SemaphoreType.DMA] * (2 * _NBUF + 1),
    )
    def _gather_kernel(table_hbm, idx_hbm, fwdbwd_hbm, out_hbm, fwd_out, bwd_out,
                       idx_v, *bufs_and_sems):
        bufs = bufs_and_sems[:_NBUF]
        gsems = bufs_and_sems[_NBUF:2 * _NBUF]
        ssems = bufs_and_sems[2 * _NBUF:3 * _NBUF]
        csem = bufs_and_sems[3 * _NBUF]
        wid = lax.axis_index("s") * _NC + lax.axis_index("c")
        base = wid * _ROWS_PER_W

        fwd_cp = pltpu.make_async_copy(fwdbwd_hbm.at[0], fwd_out, csem)
        bwd_cp = pltpu.make_async_copy(fwdbwd_hbm.at[1], bwd_out, csem)

        @pl.when(wid == 0)
        def _():
            fwd_cp.start()

        @pl.when(wid == 1)
        def _():
            bwd_cp.start()

        pltpu.sync_copy(idx_hbm.at[wid], idx_v)

        gathers = [None] * _NCHUNK
        stores = [None] * _NCHUNK
        for j in range(min(_NBUF, _NCHUNK)):
            gathers[j] = pltpu.async_copy(
                table_hbm.at[idx_v.at[j]], bufs[j], gsems[j]
            )
        for j in range(_NCHUNK):
            b = j % _NBUF
            gathers[j].wait()
            stores[j] = pltpu.async_copy(
                bufs[b], out_hbm.at[pl.ds(base + j * _CHUNK, _CHUNK)], ssems[b]
            )
            nj = j + _NBUF
            if nj < _NCHUNK:
                stores[j].wait()
                gathers[nj] = pltpu.async_copy(
                    table_hbm.at[idx_v.at[nj]], bufs[b], gsems[b]
                )
        for j in range(max(0, _NCHUNK - _NBUF), _NCHUNK):
            stores[j].wait()

        @pl.when(wid == 0)
        def _():
            fwd_cp.wait()

        @pl.when(wid == 1)
        def _():
            bwd_cp.wait()

    return _gather_kernel


def kernel(patches):
    table = patches.reshape(_T * _B, _C)
    shuffled, fwd, bwd = _build_gather_kernel()(
        table, jnp.asarray(_FLAT_IDX), jnp.asarray(_FWDBWD)
    )
    return shuffled.reshape(_REMAIN_T, _B, _C), fwd, bwd


# speedup vs baseline: 1.0725x; 1.0725x over previous
import functools

import jax
import jax.numpy as jnp
import numpy as np
from jax import lax
from jax.experimental import pallas as pl
from jax.experimental.pallas import tpu as pltpu
from jax.experimental.pallas import tpu_sc as plsc

_RATIO = 0.75
_T, _B, _C = 576, 64, 768
_REMAIN_T = int(_T * (1 - _RATIO))

_NC, _NS = 2, 16
_NW = _NC * _NS
_ROWS = _REMAIN_T * _B
_ROWS_PER_W = _ROWS // _NW
_CHUNK = 24
_NCHUNK = _ROWS_PER_W // _CHUNK
_NBUF = 6


def _make_indexes():
    with jax.default_device(jax.local_devices(backend="cpu")[0]):
        key = jax.random.key(42)
        keys = jax.random.split(key, _B)
        fwd = jnp.stack([jax.random.permutation(k, _T) for k in keys], axis=1)
        bwd = jnp.argsort(fwd, axis=0)
        return np.asarray(fwd), np.asarray(bwd)


_FWD, _BWD = _make_indexes()
_FLAT_IDX = (
    _FWD[:_REMAIN_T].astype(np.int64) * _B + np.arange(_B, dtype=np.int64)[None, :]
).astype(np.int32).reshape(_NW, _NCHUNK * _CHUNK)

@functools.lru_cache(maxsize=1)
def _build_gather_kernel():
    mesh = plsc.VectorSubcoreMesh(
        core_axis_name="c", subcore_axis_name="s",
        num_cores=_NC, num_subcores=_NS,
    )

    @functools.partial(
        pl.kernel,
        out_type=jax.ShapeDtypeStruct((_ROWS, _C), jnp.float32),
        mesh=mesh,
        scratch_types=[
            pltpu.VMEM((_NCHUNK * _CHUNK,), jnp.int32),
        ]
        + [pltpu.VMEM((_CHUNK, _C), jnp.float32)] * _NBUF
        + [pltpu.SemaphoreType.DMA] * (2 * _NBUF),
    )
    def _gather_kernel(table_hbm, idx_hbm, out_hbm, idx_v, *bufs_and_sems):
        bufs = bufs_and_sems[:_NBUF]
        gsems = bufs_and_sems[_NBUF:2 * _NBUF]
        ssems = bufs_and_sems[2 * _NBUF:]
        wid = lax.axis_index("s") * _NC + lax.axis_index("c")
        base = wid * _ROWS_PER_W
        pltpu.sync_copy(idx_hbm.at[wid], idx_v)

        gathers = [None] * _NCHUNK
        stores = [None] * _NCHUNK
        for j in range(min(_NBUF, _NCHUNK)):
            gathers[j] = pltpu.async_copy(
                table_hbm.at[idx_v.at[pl.ds(j * _CHUNK, _CHUNK)]], bufs[j], gsems[j]
            )
        for j in range(_NCHUNK):
            b = j % _NBUF
            gathers[j].wait()
            stores[j] = pltpu.async_copy(
                bufs[b], out_hbm.at[pl.ds(base + j * _CHUNK, _CHUNK)], ssems[b]
            )
            nj = j + _NBUF
            if nj < _NCHUNK:
                stores[j].wait()
                gathers[nj] = pltpu.async_copy(
                    table_hbm.at[idx_v.at[pl.ds(nj * _CHUNK, _CHUNK)]], bufs[b],
                    gsems[b]
                )
        for j in range(max(0, _NCHUNK - _NBUF), _NCHUNK):
            stores[j].wait()

    return _gather_kernel


def kernel(patches):
    table = patches.reshape(_T * _B, _C)
    out = _build_gather_kernel()(table, jnp.asarray(_FLAT_IDX))
    return out.reshape(_REMAIN_T, _B, _C), jnp.asarray(_FWD), jnp.asarray(_BWD)
